# baseline (speedup 1.0000x reference)
import functools

import jax
import jax.numpy as jnp
from jax import lax
from jax.experimental import pallas as pl
from jax.experimental.pallas import tpu as pltpu
from jax.experimental.pallas import tpu_sc as plsc

B, S, D, V = 1024, 200, 128, 100000
NC, NS = 2, 16
NW = NC * NS
BW = B // NW
PQ = 2
C = PQ * BW
CHUNKS = S // PQ
LANES = 16
CD = D // LANES
NBUF = 4
NP = 3
NPP = 8


def _matryoshka_sc(src4, E0, E1, E2, Pint):
    mesh = plsc.VectorSubcoreMesh(core_axis_name="c", subcore_axis_name="s")

    @functools.partial(
        pl.kernel,
        mesh=mesh,
        out_type=jax.ShapeDtypeStruct((B * S, D), jnp.float32),
        scratch_types=[
            pltpu.VMEM((CHUNKS, C), jnp.int32),
            pltpu.VMEM((NBUF, C, D), jnp.float32),
            pltpu.VMEM((NBUF, C, D), jnp.float32),
            pltpu.VMEM((NBUF, C, D), jnp.float32),
            pltpu.VMEM((NBUF, NPP * PQ, D), jnp.float32),
            pltpu.VMEM((NBUF, C), jnp.int32),
            pltpu.VMEM((C,), jnp.int32),
            pltpu.SemaphoreType.DMA,
            pltpu.SemaphoreType.DMA,
            pltpu.SemaphoreType.DMA,
            pltpu.SemaphoreType.DMA,
            pltpu.SemaphoreType.DMA,
            pltpu.SemaphoreType.DMA,
            pltpu.SemaphoreType.DMA,
            pltpu.SemaphoreType.DMA,
        ],
    )
    def k(src_hbm, e0, e1, e2, pint, out_hbm,
          idx_all, acc, g1, g2, pbuf, oidx, obase,
          sg0, sg1, sg2, sg3, so0, so1, so2, so3):
        semg = (sg0, sg1, sg2, sg3)
        semo = (so0, so1, so2, so3)
        wid = lax.axis_index("s") * NC + lax.axis_index("c")

        pltpu.sync_copy(src_hbm.at[wid], idx_all)

        def issue_far(ck, b):
            idx = idx_all.at[ck]
            pltpu.async_copy(e1.at[idx], g1.at[b], semg[b])
            pltpu.async_copy(e2.at[idx], g2.at[b], semg[b])
            pltpu.async_copy(pint.at[pl.ds(ck * NPP * PQ, NPP * PQ)],
                             pbuf.at[b], semg[b])

        def issue_e0(ck, b):
            pltpu.async_copy(e0.at[idx_all.at[ck]], acc.at[b], semg[b])

        def wait_chunk(b):
            for buf in (acc, g1, g2):
                pltpu.make_async_copy(e0.at[pl.ds(0, C)], buf.at[b],
                                      semg[b]).wait()
            pltpu.make_async_copy(pint.at[pl.ds(0, NPP * PQ)], pbuf.at[b],
                                  semg[b]).wait()

        def wait_store(b):
            pltpu.make_async_copy(acc.at[b], out_hbm.at[oidx.at[0]],
                                  semo[b]).wait()

        for ck in range(NBUF):
            issue_far(ck, ck)
        for ck in range(2):
            issue_e0(ck, ck)

        for v in range(C // LANES):
            q, h = divmod(v, BW // LANES)
            jvec = lax.iota(jnp.int32, LANES) + h * LANES
            obase[pl.ds(v * LANES, LANES)] = (wid * BW + jvec) * S + q

        def step(ck, b):
            eb = (b + 2) % NBUF

            @pl.when(ck >= 2)
            def _():
                wait_store(eb)

            @pl.when(ck + 2 < CHUNKS)
            def _():
                issue_e0(ck + 2, eb)

            wait_chunk(b)

            for q in range(PQ):
                pos = []
                for c in range(CD):
                    sl = pl.ds(c * LANES, LANES)
                    pos.append(pbuf[b, NPP * q, sl]
                               + pbuf[b, NPP * q + 1, sl]
                               + pbuf[b, NPP * q + 2, sl])

                def row_body(r, _pos=pos):
                    for c in range(CD):
                        sl = pl.ds(c * LANES, LANES)
                        plsc.addupdate(acc.at[b, r, sl],
                                       g1[b, r, sl] + g2[b, r, sl] + _pos[c])

                plsc.parallel_loop(q * BW, (q + 1) * BW, 1,
                                   unroll=4)(row_body)

            for v in range(C // LANES):
                sl = pl.ds(v * LANES, LANES)
                oidx[b, sl] = obase[sl] + ck * PQ
            pltpu.async_copy(acc.at[b], out_hbm.at[oidx.at[b]], semo[b])

            @pl.when(ck + NBUF < CHUNKS)
            def _():
                issue_far(ck + NBUF, b)

        def body(i, carry):
            for b in range(NBUF):
                step(i * NBUF + b, b)
            return carry

        lax.fori_loop(0, CHUNKS // NBUF, body, 0)
        wait_store(2)
        wait_store(3)

    return k(src4, E0, E1, E2, Pint)


def kernel(src, E0, E1, E2, P0, P1, P2):
    src4 = src.reshape(NW, BW, CHUNKS, PQ).transpose(0, 2, 3, 1)
    src4 = src4.reshape(NW, CHUNKS, C)
    stacked = jnp.stack([P0.reshape(-1, D)[:S], P1.reshape(-1, D)[:S],
                         P2.reshape(-1, D)[:S]], axis=1)
    Pint = jnp.pad(stacked, ((0, 0), (0, NPP - NP), (0, 0)))
    Pint = Pint.reshape(NPP * S, D)
    out = _matryoshka_sc(src4, E0, E1, E2, Pint)
    return out.reshape(B, S, D)
